# baseline (speedup 1.0000x reference)
import jax
import jax.numpy as jnp
from jax import lax
from jax.experimental import pallas as pl
from jax.experimental.pallas import tpu as pltpu
from jax.experimental.pallas import tpu_sc as plsc

_BN = 4096
_N_TOKENS = 32768
_NS = 16
_VL = 16
_NE = 8


def _gate_kernel(xa_ref, xb_ref, wt_ref, tkw_ref, tki_ref, stats_ref, acc_ref):
    step = pl.program_id(0)
    bn = xa_ref.shape[0]
    hh = xa_ref.shape[1]
    e = wt_ref.shape[1]

    @pl.when(step == 0)
    def _init():
        acc_ref[...] = jnp.zeros_like(acc_ref)

    logits = (
        jnp.dot(xa_ref[...], wt_ref[0:hh, :], preferred_element_type=jnp.float32)
        + jnp.dot(xb_ref[...], wt_ref[hh:, :], preferred_element_type=jnp.float32)
    )
    lt = logits.T

    iota = jax.lax.broadcasted_iota(jnp.int32, (e, bn), 0)
    m1 = jnp.max(lt, axis=0, keepdims=True)
    i1 = jnp.min(jnp.where(lt == m1, iota, e), axis=0, keepdims=True)
    masked = jnp.where(iota == i1, -jnp.inf, lt)
    m2 = jnp.max(masked, axis=0, keepdims=True)
    i2 = jnp.min(jnp.where(masked == m2, iota, e), axis=0, keepdims=True)

    r = jnp.exp(m2 - m1)
    denom = 1.0 + r
    tkw_ref[...] = jnp.concatenate([1.0 / denom, r / denom], axis=0)
    tki_ref[...] = jnp.concatenate([i1, i2], axis=0)

    ex = jnp.exp(lt - m1)
    probs = ex / jnp.sum(ex, axis=0, keepdims=True)
    acc_ref[...] += jnp.sum(probs, axis=1, keepdims=True)
    stats_ref[...] = acc_ref[...]


def _loss_body(idx_hbm, psum_hbm, loss_hbm,
               idx_buf, acc8_ref, cnt_ref, iota_ref, zero_ref, psum_ref,
               out_ref, shared_ref):
    cid = lax.axis_index("c")
    sid = lax.axis_index("s")
    chunk = 2 * _N_TOKENS // _NS
    iota = lax.iota(jnp.int32, _VL)

    @pl.when(sid == 0)
    def _init_shared():
        zero_ref[...] = jnp.zeros((_VL,), jnp.float32)
        pltpu.sync_copy(zero_ref, shared_ref)

    pltpu.sync_copy(idx_hbm.at[pl.ds(sid * chunk, chunk)], idx_buf)
    for e in range(_NE):
        acc8_ref[e] = jnp.zeros((_VL,), jnp.int32)
    plsc.subcore_barrier()

    one_i = jnp.ones((_VL,), jnp.int32)
    zero_i = jnp.zeros((_VL,), jnp.int32)

    def body(i, carry):
        v = idx_buf[pl.ds(i * _VL, _VL)]
        for e in range(_NE):
            acc8_ref[e] = acc8_ref[e] + jnp.where(v == e, one_i, zero_i)
        return carry

    lax.fori_loop(0, chunk // _VL, body, 0)

    cnt_ref[...] = jnp.zeros((_VL,), jnp.float32)
    for e in range(_NE):
        cum = plsc.cumsum(acc8_ref[e].astype(jnp.float32))
        plsc.store_scatter(cnt_ref,
                           [jnp.full((_VL,), e, jnp.int32)],
                           cum, mask=iota == _VL - 1)
    iota_ref[...] = iota
    pltpu.sync_copy(cnt_ref, shared_ref.at[iota_ref], add=True)
    plsc.subcore_barrier()

    @pl.when((cid == 0) & (sid == 0))
    def _finalize():
        psum_ref[...] = jnp.zeros((_VL,), jnp.float32)
        pltpu.sync_copy(psum_hbm, psum_ref.at[pl.ds(0, _NE)])
        pltpu.sync_copy(shared_ref, cnt_ref)
        scale = jnp.float32(_NE) / jnp.float32(_N_TOKENS * _N_TOKENS)
        prod = cnt_ref[...] * psum_ref[...] * scale
        out_ref[...] = plsc.cumsum(prod)
        pltpu.sync_copy(out_ref, loss_hbm)


def _loss_call(idx_flat, psum_flat):
    mesh = plsc.VectorSubcoreMesh(core_axis_name="c", subcore_axis_name="s")
    chunk = 2 * _N_TOKENS // _NS
    fn = pl.kernel(
        _loss_body,
        mesh=mesh,
        out_type=jax.ShapeDtypeStruct((_VL,), jnp.float32),
        scratch_types=[
            pltpu.VMEM((chunk,), jnp.int32),
            pltpu.VMEM((_NE, _VL), jnp.int32),
            pltpu.VMEM((_VL,), jnp.float32),
            pltpu.VMEM((_VL,), jnp.int32),
            pltpu.VMEM((_VL,), jnp.float32),
            pltpu.VMEM((_VL,), jnp.float32),
            pltpu.VMEM((_VL,), jnp.float32),
            pltpu.VMEM_SHARED((_VL,), jnp.float32),
        ],
        compiler_params=pltpu.CompilerParams(needs_layout_passes=False),
    )
    return fn(idx_flat, psum_flat)


def kernel(x, W):
    n, h = x.shape
    e = W.shape[0]
    bn = _BN
    nb = n // bn
    wt = W.T
    tkw_t, tki_t, stats = pl.pallas_call(
        _gate_kernel,
        grid=(nb,),
        in_specs=[
            pl.BlockSpec((bn, h // 2), lambda i: (i, 0)),
            pl.BlockSpec((bn, h // 2), lambda i: (i, 1)),
            pl.BlockSpec((h, e), lambda i: (0, 0)),
        ],
        out_specs=[
            pl.BlockSpec((2, bn), lambda i: (0, i)),
            pl.BlockSpec((2, bn), lambda i: (0, i)),
            pl.BlockSpec((e, 1), lambda i: (0, 0)),
        ],
        out_shape=[
            jax.ShapeDtypeStruct((2, n), jnp.float32),
            jax.ShapeDtypeStruct((2, n), jnp.int32),
            jax.ShapeDtypeStruct((e, 1), jnp.float32),
        ],
        scratch_shapes=[pltpu.VMEM((e, 1), jnp.float32)],
        compiler_params=pltpu.CompilerParams(
            dimension_semantics=("arbitrary",)),
    )(x, x, wt)
    loss_vec = _loss_call(tki_t.reshape(-1), stats.reshape(-1))
    return tkw_t.T, tki_t.T, loss_vec[_VL - 1]


# speedup vs baseline: 1.5533x; 1.5533x over previous
import jax
import jax.numpy as jnp
from jax.experimental import pallas as pl
from jax.experimental.pallas import tpu as pltpu

_BN = 2048
_N_TOKENS = 32768


def _router_kernel(xa_ref, xb_ref, wt_ref, tkw_ref, tki_ref, loss_ref, acc_ref):
    step = pl.program_id(0)
    bn = xa_ref.shape[0]
    hh = xa_ref.shape[1]
    e = wt_ref.shape[1]

    @pl.when(step == 0)
    def _init():
        acc_ref[...] = jnp.zeros_like(acc_ref)

    logits = (
        jnp.dot(xa_ref[...], wt_ref[0:hh, :], preferred_element_type=jnp.float32)
        + jnp.dot(xb_ref[...], wt_ref[hh:, :], preferred_element_type=jnp.float32)
    )
    lt = logits.T

    iota = jax.lax.broadcasted_iota(jnp.int32, (e, bn), 0)
    m1 = jnp.max(lt, axis=0, keepdims=True)
    i1 = jnp.min(jnp.where(lt == m1, iota, e), axis=0, keepdims=True)
    masked = jnp.where(iota == i1, -jnp.inf, lt)
    m2 = jnp.max(masked, axis=0, keepdims=True)
    i2 = jnp.min(jnp.where(masked == m2, iota, e), axis=0, keepdims=True)

    r = jnp.exp(m2 - m1)
    denom = 1.0 + r
    tkw_ref[...] = jnp.concatenate([1.0 / denom, r / denom], axis=0)
    tki_ref[...] = jnp.concatenate([i1, i2], axis=0)

    ex = jnp.exp(lt - m1)
    probs = ex / jnp.sum(ex, axis=0, keepdims=True)
    onehot = (iota == i1).astype(jnp.float32) + (iota == i2).astype(jnp.float32)
    acc_ref[:, 0:1] += jnp.sum(onehot, axis=1, keepdims=True)
    acc_ref[:, 1:2] += jnp.sum(probs, axis=1, keepdims=True)

    scale = jnp.float32(e) / jnp.float32(_N_TOKENS * _N_TOKENS)
    loss_ref[...] = (scale * jnp.sum(acc_ref[:, 0:1] * acc_ref[:, 1:2])).reshape(1, 1)


def kernel(x, W):
    n, h = x.shape
    e = W.shape[0]
    bn = _BN
    nb = n // bn
    wt = W.T
    tkw_t, tki_t, loss = pl.pallas_call(
        _router_kernel,
        grid=(nb,),
        in_specs=[
            pl.BlockSpec((bn, h // 2), lambda i: (i, 0)),
            pl.BlockSpec((bn, h // 2), lambda i: (i, 1)),
            pl.BlockSpec((h, e), lambda i: (0, 0)),
        ],
        out_specs=[
            pl.BlockSpec((2, bn), lambda i: (0, i)),
            pl.BlockSpec((2, bn), lambda i: (0, i)),
            pl.BlockSpec((1, 1), lambda i: (0, 0)),
        ],
        out_shape=[
            jax.ShapeDtypeStruct((2, n), jnp.float32),
            jax.ShapeDtypeStruct((2, n), jnp.int32),
            jax.ShapeDtypeStruct((1, 1), jnp.float32),
        ],
        scratch_shapes=[pltpu.VMEM((e, 2), jnp.float32)],
        compiler_params=pltpu.CompilerParams(
            dimension_semantics=("arbitrary",)),
    )(x, x, wt)
    return tkw_t.T, tki_t.T, loss.reshape(())


# speedup vs baseline: 1.6819x; 1.0828x over previous
import jax
import jax.numpy as jnp
from jax.experimental import pallas as pl
from jax.experimental.pallas import tpu as pltpu

_BN = 4096
_N_TOKENS = 32768


def _router_kernel(xa_ref, xb_ref, wt_ref, tkw_ref, tki_ref, loss_ref, acc_ref):
    step = pl.program_id(0)
    bn = xa_ref.shape[0]
    hh = xa_ref.shape[1]
    e = wt_ref.shape[1]

    @pl.when(step == 0)
    def _init():
        acc_ref[...] = jnp.zeros_like(acc_ref)

    logits = (
        jnp.dot(xa_ref[...], wt_ref[0:hh, :], preferred_element_type=jnp.float32)
        + jnp.dot(xb_ref[...], wt_ref[hh:, :], preferred_element_type=jnp.float32)
    )
    lt = logits.T

    iota = jax.lax.broadcasted_iota(jnp.int32, (e, bn), 0)
    m1 = jnp.max(lt, axis=0, keepdims=True)
    i1 = jnp.min(jnp.where(lt == m1, iota, e), axis=0, keepdims=True)
    masked = jnp.where(iota == i1, -jnp.inf, lt)
    m2 = jnp.max(masked, axis=0, keepdims=True)
    i2 = jnp.min(jnp.where(masked == m2, iota, e), axis=0, keepdims=True)

    r = jnp.exp(m2 - m1)
    denom = 1.0 + r
    tkw_ref[...] = jnp.concatenate([1.0 / denom, r / denom], axis=0)
    tki_ref[...] = jnp.concatenate([i1, i2], axis=0)

    ex = jnp.exp(lt - m1)
    probs = ex / jnp.sum(ex, axis=0, keepdims=True)
    onehot = (iota == i1).astype(jnp.float32) + (iota == i2).astype(jnp.float32)
    acc_ref[:, 0:1] += jnp.sum(onehot, axis=1, keepdims=True)
    acc_ref[:, 1:2] += jnp.sum(probs, axis=1, keepdims=True)

    scale = jnp.float32(e) / jnp.float32(_N_TOKENS * _N_TOKENS)
    loss_ref[...] = (scale * jnp.sum(acc_ref[:, 0:1] * acc_ref[:, 1:2])).reshape(1, 1)


def kernel(x, W):
    n, h = x.shape
    e = W.shape[0]
    bn = _BN
    nb = n // bn
    wt = W.T
    tkw_t, tki_t, loss = pl.pallas_call(
        _router_kernel,
        grid=(nb,),
        in_specs=[
            pl.BlockSpec((bn, h // 2), lambda i: (i, 0)),
            pl.BlockSpec((bn, h // 2), lambda i: (i, 1)),
            pl.BlockSpec((h, e), lambda i: (0, 0)),
        ],
        out_specs=[
            pl.BlockSpec((2, bn), lambda i: (0, i)),
            pl.BlockSpec((2, bn), lambda i: (0, i)),
            pl.BlockSpec((1, 1), lambda i: (0, 0)),
        ],
        out_shape=[
            jax.ShapeDtypeStruct((2, n), jnp.float32),
            jax.ShapeDtypeStruct((2, n), jnp.int32),
            jax.ShapeDtypeStruct((1, 1), jnp.float32),
        ],
        scratch_shapes=[pltpu.VMEM((e, 2), jnp.float32)],
        compiler_params=pltpu.CompilerParams(
            dimension_semantics=("arbitrary",)),
    )(x, x, wt)
    return tkw_t.T, tki_t.T, loss.reshape(())


# speedup vs baseline: 1.6963x; 1.0086x over previous
import jax
import jax.numpy as jnp
from jax.experimental import pallas as pl
from jax.experimental.pallas import tpu as pltpu

_BN = 4096
_N_TOKENS = 32768


def _router_kernel(xa_ref, xb_ref, wt_ref, tkw_ref, tki_ref, loss_ref, acc_ref):
    step = pl.program_id(0)
    bn = 2 * xa_ref.shape[0]
    e = wt_ref.shape[1]

    @pl.when(step == 0)
    def _init():
        acc_ref[...] = jnp.zeros_like(acc_ref)

    logits = jnp.concatenate(
        [jnp.dot(xa_ref[...], wt_ref[...], preferred_element_type=jnp.float32),
         jnp.dot(xb_ref[...], wt_ref[...], preferred_element_type=jnp.float32)],
        axis=0)
    lt = logits.T

    iota = jax.lax.broadcasted_iota(jnp.int32, (e, bn), 0)
    m1 = jnp.max(lt, axis=0, keepdims=True)
    i1 = jnp.min(jnp.where(lt == m1, iota, e), axis=0, keepdims=True)
    masked = jnp.where(iota == i1, -jnp.inf, lt)
    m2 = jnp.max(masked, axis=0, keepdims=True)
    i2 = jnp.min(jnp.where(masked == m2, iota, e), axis=0, keepdims=True)

    r = jnp.exp(m2 - m1)
    denom = 1.0 + r
    tkw_ref[...] = jnp.concatenate([1.0 / denom, r / denom], axis=0)
    tki_ref[...] = jnp.concatenate([i1, i2], axis=0)

    ex = jnp.exp(lt - m1)
    probs = ex / jnp.sum(ex, axis=0, keepdims=True)
    onehot = (iota == i1).astype(jnp.float32) + (iota == i2).astype(jnp.float32)
    acc_ref[:, 0:1] += jnp.sum(onehot, axis=1, keepdims=True)
    acc_ref[:, 1:2] += jnp.sum(probs, axis=1, keepdims=True)

    scale = jnp.float32(e) / jnp.float32(_N_TOKENS * _N_TOKENS)
    loss_ref[...] = (scale * jnp.sum(acc_ref[:, 0:1] * acc_ref[:, 1:2])).reshape(1, 1)


def kernel(x, W):
    n, h = x.shape
    e = W.shape[0]
    bn = _BN
    nb = n // bn
    wt = W.T
    tkw_t, tki_t, loss = pl.pallas_call(
        _router_kernel,
        grid=(nb,),
        in_specs=[
            pl.BlockSpec((bn // 2, h), lambda i: (2 * i, 0)),
            pl.BlockSpec((bn // 2, h), lambda i: (2 * i + 1, 0)),
            pl.BlockSpec((h, e), lambda i: (0, 0)),
        ],
        out_specs=[
            pl.BlockSpec((2, bn), lambda i: (0, i)),
            pl.BlockSpec((2, bn), lambda i: (0, i)),
            pl.BlockSpec((1, 1), lambda i: (0, 0)),
        ],
        out_shape=[
            jax.ShapeDtypeStruct((2, n), jnp.float32),
            jax.ShapeDtypeStruct((2, n), jnp.int32),
            jax.ShapeDtypeStruct((1, 1), jnp.float32),
        ],
        scratch_shapes=[pltpu.VMEM((e, 2), jnp.float32)],
        compiler_params=pltpu.CompilerParams(
            dimension_semantics=("arbitrary",)),
    )(x, x, wt)
    return tkw_t.T, tki_t.T, loss.reshape(())
